# baseline (speedup 1.0000x reference)
import jax
import jax.numpy as jnp
from jax.experimental import pallas as pl
from jax.experimental.pallas import tpu as pltpu

_LANE = 512
_TILE_ROWS = 512


def _fma_kernel(w_ref, b_ref, x_ref, o_ref):
    o_ref[...] = x_ref[...] * w_ref[0, 0] + b_ref[0]


def kernel(x, weight, bias):
    n, f = x.shape
    assert f == 1

    n_main = (n // _LANE) * _LANE
    if n_main == 0:
        return x * weight.reshape(()) + bias.reshape(())

    rows = n_main // _LANE
    tile_r = min(_TILE_ROWS, rows)
    grid = (pl.cdiv(rows, tile_r),)

    x_flat = x.reshape(-1)
    x2d = (x_flat if n_main == n else x_flat[:n_main]).reshape(rows, _LANE)

    y2d = pl.pallas_call(
        _fma_kernel,
        out_shape=jax.ShapeDtypeStruct((rows, _LANE), x.dtype),
        grid=grid,
        in_specs=[
            pl.BlockSpec(memory_space=pltpu.SMEM),
            pl.BlockSpec(memory_space=pltpu.SMEM),
            pl.BlockSpec((tile_r, _LANE), lambda i: (i, 0)),
        ],
        out_specs=pl.BlockSpec((tile_r, _LANE), lambda i: (i, 0)),
        compiler_params=pltpu.CompilerParams(
            dimension_semantics=("parallel",),
        ),
    )(weight, bias, x2d)

    y_flat = y2d.reshape(n_main)
    if n_main != n:
        y_tail = x_flat[n_main:] * weight.reshape(()) + bias.reshape(())
        y_flat = jnp.concatenate([y_flat, y_tail])
    return y_flat.reshape(n, 1)


# speedup vs baseline: 14.1211x; 14.1211x over previous
import jax
import jax.numpy as jnp
from jax.experimental import pallas as pl
from jax.experimental.pallas import tpu as pltpu

_CHUNK = 512 * 1024


def _fma_kernel(w_ref, b_ref, x_ref, o_ref):
    o_ref[...] = x_ref[...] * w_ref[0, 0] + b_ref[0]


def kernel(x, weight, bias):
    n, f = x.shape
    assert f == 1

    x1 = x.reshape(n)
    grid = (pl.cdiv(n, _CHUNK),)

    y1 = pl.pallas_call(
        _fma_kernel,
        out_shape=jax.ShapeDtypeStruct((n,), x.dtype),
        grid=grid,
        in_specs=[
            pl.BlockSpec(memory_space=pltpu.SMEM),
            pl.BlockSpec(memory_space=pltpu.SMEM),
            pl.BlockSpec((_CHUNK,), lambda i: (i,)),
        ],
        out_specs=pl.BlockSpec((_CHUNK,), lambda i: (i,)),
        compiler_params=pltpu.CompilerParams(
            dimension_semantics=("parallel",),
        ),
    )(weight, bias, x1)

    return y1.reshape(n, 1)


# speedup vs baseline: 15.4965x; 1.0974x over previous
import jax
import jax.numpy as jnp
from jax.experimental import pallas as pl
from jax.experimental.pallas import tpu as pltpu

_CHUNK = 1024 * 1024


def _fma_kernel(w_ref, b_ref, x_ref, o_ref):
    o_ref[...] = x_ref[...] * w_ref[0, 0] + b_ref[0]


def kernel(x, weight, bias):
    n, f = x.shape
    assert f == 1

    x1 = x.reshape(n)
    grid = (pl.cdiv(n, _CHUNK),)

    y1 = pl.pallas_call(
        _fma_kernel,
        out_shape=jax.ShapeDtypeStruct((n,), x.dtype),
        grid=grid,
        in_specs=[
            pl.BlockSpec(memory_space=pltpu.SMEM),
            pl.BlockSpec(memory_space=pltpu.SMEM),
            pl.BlockSpec((_CHUNK,), lambda i: (i,)),
        ],
        out_specs=pl.BlockSpec((_CHUNK,), lambda i: (i,)),
        compiler_params=pltpu.CompilerParams(
            dimension_semantics=("parallel",),
        ),
    )(weight, bias, x1)

    return y1.reshape(n, 1)


# speedup vs baseline: 16.5678x; 1.0691x over previous
import jax
import jax.numpy as jnp
from jax.experimental import pallas as pl
from jax.experimental.pallas import tpu as pltpu

_CHUNK = 2048 * 1024


def _fma_kernel(w_ref, b_ref, x_ref, o_ref):
    o_ref[...] = x_ref[...] * w_ref[0, 0] + b_ref[0]


def kernel(x, weight, bias):
    n, f = x.shape
    assert f == 1

    x1 = x.reshape(n)
    grid = (pl.cdiv(n, _CHUNK),)

    y1 = pl.pallas_call(
        _fma_kernel,
        out_shape=jax.ShapeDtypeStruct((n,), x.dtype),
        grid=grid,
        in_specs=[
            pl.BlockSpec(memory_space=pltpu.SMEM),
            pl.BlockSpec(memory_space=pltpu.SMEM),
            pl.BlockSpec((_CHUNK,), lambda i: (i,)),
        ],
        out_specs=pl.BlockSpec((_CHUNK,), lambda i: (i,)),
        compiler_params=pltpu.CompilerParams(
            dimension_semantics=("parallel",),
        ),
    )(weight, bias, x1)

    return y1.reshape(n, 1)
